# baseline (speedup 1.0000x reference)
import functools

import jax
import jax.numpy as jnp
import numpy as np
from jax import lax
from jax.experimental import pallas as pl
from jax.experimental.pallas import tpu as pltpu
from jax.experimental.pallas import tpu_sc as plsc

_ROWS = 16384
_COLS = 1024
_TOT = _ROWS * _COLS
_BINS = 10
_BLOCK_ROWS = 1024
_NBLOCKS = _ROWS // _BLOCK_ROWS

_SC_BLOCKS = 6
_SC_ROWS = _SC_BLOCKS * _BLOCK_ROWS

_NW = 32
_ROWS_PW = _SC_ROWS // _NW
_CHR = 16
_NCH = _ROWS_PW // _CHR
_CH = _CHR * _COLS
_POPCNT_BINS = 0

_EDGE_LOGITS = [
    np.float32(np.log(np.float64(np.float32(b / 10.0))
                      / (1.0 - np.float64(np.float32(b / 10.0)))))
    for b in range(1, _BINS)
]


def _sc_body(pred_hbm, tgt_hbm, out_hbm, p0, p1, t0, t1, obuf,
             sp0, sp1, st0, st1):
    wid = lax.axis_index("s") * 2 + lax.axis_index("c")
    base = wid * _ROWS_PW
    pbufs, tbufs = (p0, p1), (t0, t1)
    psems, tsems = (sp0, sp1), (st0, st1)

    def _copies(chunk, slot):
        row = base + chunk * _CHR
        out = []
        for r in range(_CHR):
            out.append(pltpu.make_async_copy(
                pred_hbm.at[row + r],
                pbufs[slot].at[pl.ds(r * _COLS, _COLS)], psems[slot]))
            out.append(pltpu.make_async_copy(
                tgt_hbm.at[row + r],
                tbufs[slot].at[pl.ds(r * _COLS, _COLS)], tsems[slot]))
        return out

    for slot in range(2):
        for c in _copies(slot, slot):
            c.start()

    def _process(slot, accs):
        pb, tb = pbufs[slot], tbufs[slot]

        def vec(v, accs):
            new = list(accs)
            for u in range(4):
                off = (v * 4 + u) * 16
                p = pb[pl.ds(off, 16)]
                t = tb[pl.ds(off, 16)]
                q = jnp.where(t > 0.5, -p, p)
                for b in range(_BINS - 1):
                    m = q < _EDGE_LOGITS[b]
                    if b < _POPCNT_BINS:
                        new[b] = new[b] + plsc.all_reduce_population_count(m)
                    else:
                        new[b] = new[b] + jnp.where(m, 1, 0)
            return tuple(new)

        return lax.fori_loop(0, _CH // 64, vec, accs)

    def group(gi, accs):
        for slot in range(2):
            chunk = 2 * gi + slot
            for c in _copies(chunk, slot):
                c.wait()
            accs = _process(slot, accs)

            @pl.when(chunk + 2 < _NCH)
            def _issue():
                for c in _copies(chunk + 2, slot):
                    c.start()
        return accs

    zero = jnp.zeros((16,), jnp.int32)
    accs = lax.fori_loop(0, _NCH // 2, group, (zero,) * (_BINS - 1))
    for b in range(_BINS - 1):
        obuf[pl.ds(16 * b, 16)] = accs[b]
    pltpu.sync_copy(obuf, out_hbm.at[wid])


_sc_counts = functools.partial(
    pl.kernel,
    out_type=jax.ShapeDtypeStruct((_NW, (_BINS - 1) * 16), jnp.int32),
    mesh=plsc.VectorSubcoreMesh(core_axis_name="c", subcore_axis_name="s"),
    scratch_types=[
        pltpu.VMEM((_CH,), jnp.float32),
        pltpu.VMEM((_CH,), jnp.float32),
        pltpu.VMEM((_CH,), jnp.float32),
        pltpu.VMEM((_CH,), jnp.float32),
        pltpu.VMEM(((_BINS - 1) * 16,), jnp.int32),
        pltpu.SemaphoreType.DMA,
        pltpu.SemaphoreType.DMA,
        pltpu.SemaphoreType.DMA,
        pltpu.SemaphoreType.DMA,
    ],
)(_sc_body)


def _tc_body(pred_ref, target_ref, out_ref):
    i = pl.program_id(0)

    @pl.when(i == 0)
    def _init():
        for r in range(2):
            for b in range(_BINS):
                out_ref[r, b] = jnp.float32(0.0)

    p = pred_ref[...]
    t = target_ref[...]
    u = jnp.exp(-jnp.abs(p))
    loss = jnp.maximum(p, 0.0) - p * t + jnp.log1p(u)
    q = jnp.where(t > 0.5, -p, p)

    inds = [jnp.where(q < _EDGE_LOGITS[b], 1.0, 0.0)
            for b in range(_BINS - 1)]
    for b in range(_BINS - 1):
        out_ref[1, b] += jnp.sum(loss * inds[b])
    out_ref[1, _BINS - 1] += jnp.sum(loss)

    @pl.when(i >= _SC_BLOCKS)
    def _counts():
        for b in range(_BINS - 1):
            out_ref[0, b] += jnp.sum(inds[b])


def _tc_call(pred, target):
    return pl.pallas_call(
        _tc_body,
        grid=(_NBLOCKS,),
        in_specs=[
            pl.BlockSpec((_BLOCK_ROWS, _COLS), lambda i: (i, 0)),
            pl.BlockSpec((_BLOCK_ROWS, _COLS), lambda i: (i, 0)),
        ],
        out_specs=pl.BlockSpec(memory_space=pltpu.SMEM),
        out_shape=jax.ShapeDtypeStruct((2, _BINS), jnp.float32),
        compiler_params=pltpu.CompilerParams(
            dimension_semantics=("arbitrary",),
        ),
    )(pred, target)


@functools.partial(jax.jit)
def kernel(pred, target):
    sc_cnt = _sc_counts(pred, target)
    tc = _tc_call(pred, target)

    ccum_sc = jnp.sum(sc_cnt.reshape(_NW, _BINS - 1, 16), axis=(0, 2))
    div = jnp.array([16] * _POPCNT_BINS
                    + [1] * (_BINS - 1 - _POPCNT_BINS), jnp.int32)
    ccum_sc = ccum_sc // div
    ccum = ccum_sc.astype(jnp.float32) + tc[0, :_BINS - 1]
    ccum = jnp.concatenate([ccum, jnp.array([np.float32(_TOT)])])
    scum = tc[1]
    zero1 = jnp.zeros((1,), jnp.float32)
    c = jnp.diff(jnp.concatenate([zero1, ccum]))
    s = jnp.diff(jnp.concatenate([zero1, scum]))
    nonempty = c > 0.0
    n = jnp.sum(nonempty.astype(jnp.float32))
    loss = jnp.sum(jnp.where(nonempty, s / jnp.where(nonempty, c, 1.0), 0.0))
    return loss / n


# speedup vs baseline: 1.0286x; 1.0286x over previous
import functools

import jax
import jax.numpy as jnp
import numpy as np
from jax import lax
from jax.experimental import pallas as pl
from jax.experimental.pallas import tpu as pltpu
from jax.experimental.pallas import tpu_sc as plsc

_ROWS = 16384
_COLS = 1024
_TOT = _ROWS * _COLS
_BINS = 10
_BLOCK_ROWS = 1024
_NBLOCKS = _ROWS // _BLOCK_ROWS

_SC_BLOCKS = 5
_SC_ROWS = _SC_BLOCKS * _BLOCK_ROWS

_NW = 32
_ROWS_PW = _SC_ROWS // _NW
_CHR = 16
_NCH = _ROWS_PW // _CHR
_CH = _CHR * _COLS
_POPCNT_BINS = 0

_EDGE_LOGITS = [
    np.float32(np.log(np.float64(np.float32(b / 10.0))
                      / (1.0 - np.float64(np.float32(b / 10.0)))))
    for b in range(1, _BINS)
]


def _sc_body(pred_hbm, tgt_hbm, out_hbm, p0, p1, t0, t1, obuf,
             sp0, sp1, st0, st1):
    wid = lax.axis_index("s") * 2 + lax.axis_index("c")
    base = wid * _ROWS_PW
    pbufs, tbufs = (p0, p1), (t0, t1)
    psems, tsems = (sp0, sp1), (st0, st1)

    def _copies(chunk, slot):
        row = base + chunk * _CHR
        out = []
        for r in range(_CHR):
            out.append(pltpu.make_async_copy(
                pred_hbm.at[row + r],
                pbufs[slot].at[pl.ds(r * _COLS, _COLS)], psems[slot]))
            out.append(pltpu.make_async_copy(
                tgt_hbm.at[row + r],
                tbufs[slot].at[pl.ds(r * _COLS, _COLS)], tsems[slot]))
        return out

    for slot in range(2):
        for c in _copies(slot, slot):
            c.start()

    def _process(slot, accs):
        pb, tb = pbufs[slot], tbufs[slot]

        def vec(v, accs):
            new = list(accs)
            for u in range(4):
                off = (v * 4 + u) * 16
                p = pb[pl.ds(off, 16)]
                t = tb[pl.ds(off, 16)]
                q = jnp.where(t > 0.5, -p, p)
                for b in range(_BINS - 1):
                    m = q < _EDGE_LOGITS[b]
                    if b < _POPCNT_BINS:
                        new[b] = new[b] + plsc.all_reduce_population_count(m)
                    else:
                        new[b] = new[b] + jnp.where(m, 1, 0)
            return tuple(new)

        return lax.fori_loop(0, _CH // 64, vec, accs)

    def group(gi, accs):
        for slot in range(2):
            chunk = 2 * gi + slot
            for c in _copies(chunk, slot):
                c.wait()
            accs = _process(slot, accs)

            @pl.when(chunk + 2 < _NCH)
            def _issue():
                for c in _copies(chunk + 2, slot):
                    c.start()
        return accs

    zero = jnp.zeros((16,), jnp.int32)
    accs = lax.fori_loop(0, _NCH // 2, group, (zero,) * (_BINS - 1))
    for b in range(_BINS - 1):
        obuf[pl.ds(16 * b, 16)] = accs[b]
    pltpu.sync_copy(obuf, out_hbm.at[wid])


_sc_counts = functools.partial(
    pl.kernel,
    out_type=jax.ShapeDtypeStruct((_NW, (_BINS - 1) * 16), jnp.int32),
    mesh=plsc.VectorSubcoreMesh(core_axis_name="c", subcore_axis_name="s"),
    scratch_types=[
        pltpu.VMEM((_CH,), jnp.float32),
        pltpu.VMEM((_CH,), jnp.float32),
        pltpu.VMEM((_CH,), jnp.float32),
        pltpu.VMEM((_CH,), jnp.float32),
        pltpu.VMEM(((_BINS - 1) * 16,), jnp.int32),
        pltpu.SemaphoreType.DMA,
        pltpu.SemaphoreType.DMA,
        pltpu.SemaphoreType.DMA,
        pltpu.SemaphoreType.DMA,
    ],
)(_sc_body)


def _tc_body(pred_ref, target_ref, out_ref):
    i = pl.program_id(0)

    @pl.when(i == 0)
    def _init():
        for r in range(2):
            for b in range(_BINS):
                out_ref[r, b] = jnp.float32(0.0)

    p = pred_ref[...]
    t = target_ref[...]
    u = jnp.exp(-jnp.abs(p))
    loss = jnp.maximum(p, 0.0) - p * t + jnp.log1p(u)
    q = jnp.where(t > 0.5, -p, p)

    inds = [jnp.where(q < _EDGE_LOGITS[b], 1.0, 0.0)
            for b in range(_BINS - 1)]
    for b in range(_BINS - 1):
        out_ref[1, b] += jnp.sum(loss * inds[b])
    out_ref[1, _BINS - 1] += jnp.sum(loss)

    @pl.when(i >= _SC_BLOCKS)
    def _counts():
        for b in range(_BINS - 1):
            out_ref[0, b] += jnp.sum(inds[b])


def _tc_call(pred, target):
    return pl.pallas_call(
        _tc_body,
        grid=(_NBLOCKS,),
        in_specs=[
            pl.BlockSpec((_BLOCK_ROWS, _COLS), lambda i: (i, 0)),
            pl.BlockSpec((_BLOCK_ROWS, _COLS), lambda i: (i, 0)),
        ],
        out_specs=pl.BlockSpec(memory_space=pltpu.SMEM),
        out_shape=jax.ShapeDtypeStruct((2, _BINS), jnp.float32),
        compiler_params=pltpu.CompilerParams(
            dimension_semantics=("arbitrary",),
        ),
    )(pred, target)


@functools.partial(jax.jit)
def kernel(pred, target):
    sc_cnt = _sc_counts(pred, target)
    tc = _tc_call(pred, target)

    ccum_sc = jnp.sum(sc_cnt.reshape(_NW, _BINS - 1, 16), axis=(0, 2))
    div = jnp.array([16] * _POPCNT_BINS
                    + [1] * (_BINS - 1 - _POPCNT_BINS), jnp.int32)
    ccum_sc = ccum_sc // div
    ccum = ccum_sc.astype(jnp.float32) + tc[0, :_BINS - 1]
    ccum = jnp.concatenate([ccum, jnp.array([np.float32(_TOT)])])
    scum = tc[1]
    zero1 = jnp.zeros((1,), jnp.float32)
    c = jnp.diff(jnp.concatenate([zero1, ccum]))
    s = jnp.diff(jnp.concatenate([zero1, scum]))
    nonempty = c > 0.0
    n = jnp.sum(nonempty.astype(jnp.float32))
    loss = jnp.sum(jnp.where(nonempty, s / jnp.where(nonempty, c, 1.0), 0.0))
    return loss / n
